# Initial kernel scaffold; baseline (speedup 1.0000x reference)
import jax
import jax.numpy as jnp
from jax.experimental import pallas as pl


def kernel(x_sequence, edge_index, batch, Wz, bz, Wr, br, Wc, bc, gamma, beta):
    raise NotImplementedError("write your pallas kernel here")


# speedup vs baseline: 7.1238x; 7.1238x over previous
import functools

import jax
import jax.numpy as jnp
from jax import lax
from jax.experimental import pallas as pl
from jax.experimental.pallas import tpu as pltpu
from jax.experimental.pallas import tpu_sc as plsc

N = 10000
NB = 2
NT = NB * N
E1 = 320000
EB = NB * E1
F = 128
T = 4
NSC = 2
NTILE = 16
CH = 128
RB = 2000

_GROUPS = N // 8
_GBASE = _GROUPS // NTILE
_GEXTRA = _GROUPS - _GBASE * NTILE
_ROWS_MAIN = 8 * _GBASE


@functools.lru_cache(maxsize=None)
def _mesh():
  return plsc.VectorSubcoreMesh(
      core_axis_name="c", subcore_axis_name="s", num_cores=NSC,
      num_subcores=NTILE)


def _zero_stage(stage, nrows):
  zero16 = jnp.zeros((16,), jnp.float32)
  ncol = stage.shape[1] // 16

  def zrow(i, carry):
    for k in range(ncol):
      stage[i, pl.ds(k * 16, 16)] = zero16
    return carry

  lax.fori_loop(0, nrows, zrow, 0)


def _each_acc_slice(s, fn):
  row0 = 8 * (s * _GBASE + jnp.minimum(s, _GEXTRA))
  fn(row0, _ROWS_MAIN)

  @pl.when(s < _GEXTRA)
  def _():
    fn(row0 + _ROWS_MAIN, 8)


def _zero_acc(acc, stage, s):
  def zero_slice(row0, nrows):
    nfull = nrows // CH
    rem = nrows - nfull * CH
    for j in range(nfull):
      pltpu.sync_copy(stage.at[pl.ds(0, CH)],
                      acc.at[pl.ds(row0 + j * CH, CH)])
    if rem:
      pltpu.sync_copy(stage.at[pl.ds(0, rem)],
                      acc.at[pl.ds(row0 + nfull * CH, rem)])

  _each_acc_slice(s, zero_slice)


def _chunk_span(s, total_chunks):
  base = total_chunks // NTILE
  extra = total_chunks - base * NTILE
  nch = jnp.where(s < extra, base + 1, base)
  chunk0 = s * base + jnp.minimum(s, extra)
  return chunk0, nch


def _sc_agg_body(p_hbm, src_hbm, dst_hbm, out_hbm, sidx, didx, stage, acc):
  c = lax.axis_index("c")
  s = lax.axis_index("s")

  _zero_stage(stage, CH)
  _zero_acc(acc, stage, s)
  plsc.subcore_barrier()

  chunk0, nch = _chunk_span(s, E1 // CH)
  dbase = c * N

  def chunk(j, carry):
    eb = (c * E1 // CH + chunk0 + j) * CH
    pltpu.sync_copy(src_hbm.at[pl.ds(eb, CH)], sidx)
    pltpu.sync_copy(dst_hbm.at[pl.ds(eb, CH)], didx)
    for k in range(CH // 16):
      didx[pl.ds(k * 16, 16)] = didx[pl.ds(k * 16, 16)] - dbase
    pltpu.sync_copy(p_hbm.at[sidx], stage)
    pltpu.sync_copy(stage, acc.at[didx], add=True)
    return carry

  lax.fori_loop(0, nch, chunk, 0)
  plsc.subcore_barrier()

  def writeout(row0, nrows):
    pltpu.sync_copy(acc.at[pl.ds(row0, nrows)],
                    out_hbm.at[pl.ds(dbase + row0, nrows)])

  _each_acc_slice(s, writeout)


@functools.lru_cache(maxsize=None)
def _sc_agg_kernel():
  return pl.kernel(
      _sc_agg_body,
      out_type=jax.ShapeDtypeStruct((NT, F), jnp.float32),
      mesh=_mesh(),
      scratch_types=[
          pltpu.VMEM((CH,), jnp.int32),
          pltpu.VMEM((CH,), jnp.int32),
          pltpu.VMEM((CH, F), jnp.float32),
          pltpu.VMEM_SHARED((N, F), jnp.float32),
      ],
  )


def _sc_agg(p, src, dst):
  return _sc_agg_kernel()(p, src, dst)


def _sc_degree_body(dst_hbm, out_hbm, didx, stage, acc):
  c = lax.axis_index("c")
  s = lax.axis_index("s")

  _zero_stage(stage, CH)
  _zero_acc(acc, stage, s)

  one0 = jnp.where(lax.iota(jnp.int32, 16) == 0, 1.0, 0.0).astype(jnp.float32)

  def orow(i, carry):
    stage[i, pl.ds(0, 16)] = one0
    return carry

  lax.fori_loop(0, CH, orow, 0)
  plsc.subcore_barrier()

  chunk0, nch = _chunk_span(s, E1 // CH)
  dbase = c * N

  def chunk(j, carry):
    eb = (c * E1 // CH + chunk0 + j) * CH
    pltpu.sync_copy(dst_hbm.at[pl.ds(eb, CH)], didx)
    for k in range(CH // 16):
      didx[pl.ds(k * 16, 16)] = didx[pl.ds(k * 16, 16)] - dbase
    pltpu.sync_copy(stage, acc.at[didx], add=True)
    return carry

  lax.fori_loop(0, nch, chunk, 0)
  plsc.subcore_barrier()

  def writeout(row0, nrows):
    pltpu.sync_copy(acc.at[pl.ds(row0, nrows)],
                    out_hbm.at[pl.ds(dbase + row0, nrows)])

  _each_acc_slice(s, writeout)


@functools.lru_cache(maxsize=None)
def _sc_degree_kernel():
  return pl.kernel(
      _sc_degree_body,
      out_type=jax.ShapeDtypeStruct((NT, 16), jnp.float32),
      mesh=_mesh(),
      scratch_types=[
          pltpu.VMEM((CH,), jnp.int32),
          pltpu.VMEM((CH, 16), jnp.float32),
          pltpu.VMEM_SHARED((N, 16), jnp.float32),
      ],
  )


def _sc_degree(dst):
  return _sc_degree_kernel()(dst)


def _prep_body(deg_ref, x_ref, dinv_ref, px_ref):
  d = deg_ref[:, 0:1] + 1.0
  dinvb = jnp.broadcast_to(lax.rsqrt(d), (RB, F))
  dinv_ref[...] = dinvb
  px_ref[...] = x_ref[...] * dinvb[None]


def _prep(degs, x_r):
  return pl.pallas_call(
      _prep_body,
      grid=(NT // RB,),
      in_specs=[
          pl.BlockSpec((RB, 16), lambda i: (i, 0)),
          pl.BlockSpec((T, RB, F), lambda i: (0, i, 0)),
      ],
      out_specs=[
          pl.BlockSpec((RB, F), lambda i: (i, 0)),
          pl.BlockSpec((T, RB, F), lambda i: (0, i, 0)),
      ],
      out_shape=[
          jax.ShapeDtypeStruct((NT, F), jnp.float32),
          jax.ShapeDtypeStruct((T, NT, F), jnp.float32),
      ],
  )(degs, x_r)


def _gates_body(sx_ref, sh_ref, px_ref, ph_ref, h_ref, dinv_ref, wz_ref,
                bz_ref, wr_ref, br_ref, z_ref, qrh_ref, aggx_ref):
  dinv = dinv_ref[...]
  aggx = dinv * (sx_ref[...] + px_ref[...])
  aggh = dinv * (sh_ref[...] + ph_ref[...])
  wz = wz_ref[...]
  wr = wr_ref[...]
  za = (jnp.dot(aggx, wz[:F], preferred_element_type=jnp.float32)
        + jnp.dot(aggh, wz[F:], preferred_element_type=jnp.float32)
        + bz_ref[...])
  ra = (jnp.dot(aggx, wr[:F], preferred_element_type=jnp.float32)
        + jnp.dot(aggh, wr[F:], preferred_element_type=jnp.float32)
        + br_ref[...])
  z = jax.nn.sigmoid(za)
  r = jax.nn.sigmoid(ra)
  z_ref[...] = z
  qrh_ref[...] = dinv * (r * h_ref[...])
  aggx_ref[...] = aggx


def _gates(sx, sh, px, ph, h, dinv_b, wz, bz, wr, br):
  row = pl.BlockSpec((RB, F), lambda i: (i, 0))
  wspec = pl.BlockSpec((2 * F, F), lambda i: (0, 0))
  bspec = pl.BlockSpec((1, F), lambda i: (0, 0))
  return pl.pallas_call(
      _gates_body,
      grid=(NT // RB,),
      in_specs=[row, row, row, row, row, row, wspec, bspec, wspec, bspec],
      out_specs=[row, row, row],
      out_shape=[jax.ShapeDtypeStruct((NT, F), jnp.float32)] * 3,
  )(sx, sh, px, ph, h, dinv_b, wz, bz, wr, br)


def _update_body(srh_ref, qrh_ref, aggx_ref, z_ref, h_ref, dinv_ref, wc_ref,
                 bc_ref, gamma_ref, beta_ref, h_out, ph_out):
  dinv = dinv_ref[...]
  aggrh = dinv * (srh_ref[...] + qrh_ref[...])
  wc = wc_ref[...]
  ca = (jnp.dot(aggx_ref[...], wc[:F], preferred_element_type=jnp.float32)
        + jnp.dot(aggrh, wc[F:], preferred_element_type=jnp.float32)
        + bc_ref[...])
  cand = jnp.tanh(ca)
  z = z_ref[...]
  hn = (1.0 - z) * h_ref[...] + z * cand
  mu = jnp.mean(hn, axis=1, keepdims=True)
  var = jnp.mean((hn - mu) ** 2, axis=1, keepdims=True)
  hln = (hn - mu) * lax.rsqrt(var + 1e-5) * gamma_ref[...] + beta_ref[...]
  h_out[...] = hln
  ph_out[...] = dinv * hln


def _update(srh, qrh, aggx, z, h, dinv_b, wc, bc, gamma, beta):
  row = pl.BlockSpec((RB, F), lambda i: (i, 0))
  wspec = pl.BlockSpec((2 * F, F), lambda i: (0, 0))
  bspec = pl.BlockSpec((1, F), lambda i: (0, 0))
  return pl.pallas_call(
      _update_body,
      grid=(NT // RB,),
      in_specs=[row, row, row, row, row, row, wspec, bspec, bspec, bspec],
      out_specs=[row, row],
      out_shape=[jax.ShapeDtypeStruct((NT, F), jnp.float32)] * 2,
  )(srh, qrh, aggx, z, h, dinv_b, wc, bc, gamma, beta)


def _pool_body(h_ref, out_ref):
  i = pl.program_id(0)

  @pl.when(i == 0)
  def _():
    out_ref[...] = jnp.zeros_like(out_ref)

  ssum = jnp.sum(h_ref[...], axis=0, keepdims=True) * (1.0 / N)
  blocks_per_sample = N // RB
  out_ref[0:1] += jnp.where(i < blocks_per_sample, ssum, 0.0)
  out_ref[1:2] += jnp.where(i >= blocks_per_sample, ssum, 0.0)


def _pool(h):
  return pl.pallas_call(
      _pool_body,
      grid=(NT // RB,),
      in_specs=[pl.BlockSpec((RB, F), lambda i: (i, 0))],
      out_specs=pl.BlockSpec((8, F), lambda i: (0, 0)),
      out_shape=jax.ShapeDtypeStruct((8, F), jnp.float32),
  )(h)


def kernel(x_sequence, edge_index, batch, Wz, bz, Wr, br, Wc, bc, gamma,
           beta):
  del batch
  ei = edge_index.astype(jnp.int32)
  src = jnp.concatenate([ei[0], ei[0] + N])
  dst = jnp.concatenate([ei[1], ei[1] + N])
  x_r = jnp.transpose(x_sequence, (1, 0, 2, 3)).reshape(T, NT, F)

  degs = _sc_degree(dst)
  dinv_b, px = _prep(degs, x_r)

  bz2 = bz.reshape(1, F)
  br2 = br.reshape(1, F)
  bc2 = bc.reshape(1, F)
  gamma2 = gamma.reshape(1, F)
  beta2 = beta.reshape(1, F)

  h = jnp.zeros((NT, F), jnp.float32)
  ph = jnp.zeros((NT, F), jnp.float32)
  sh = jnp.zeros((NT, F), jnp.float32)
  for t in range(T):
    pxt = px[t]
    sxt = _sc_agg(pxt, src, dst)
    z, qrh, aggx = _gates(sxt, sh, pxt, ph, h, dinv_b, Wz, bz2, Wr, br2)
    srh = _sc_agg(qrh, src, dst)
    h, ph = _update(srh, qrh, aggx, z, h, dinv_b, Wc, bc2, gamma2, beta2)
    if t < T - 1:
      sh = _sc_agg(ph, src, dst)
  return _pool(h)[:NB]


# speedup vs baseline: 12.0778x; 1.6954x over previous
import functools

import jax
import jax.numpy as jnp
from jax import lax
from jax.experimental import pallas as pl
from jax.experimental.pallas import tpu as pltpu
from jax.experimental.pallas import tpu_sc as plsc

N = 10000
NB = 2
NT = NB * N
E1 = 320000
EB = NB * E1
F = 128
T = 4
NSC = 2
NTILE = 16
CH = 128
RB = 2000
KBUF = 3
CPT = 162
E1P = CPT * NTILE * CH
PAD = E1P - E1
NGROUP = CPT // KBUF
NACC = N + 8


@functools.lru_cache(maxsize=None)
def _mesh():
  return plsc.VectorSubcoreMesh(
      core_axis_name="c", subcore_axis_name="s", num_cores=NSC,
      num_subcores=NTILE)


def _zero_stage(stage, nrows):
  zero16 = jnp.zeros((16,), jnp.float32)
  ncol = stage.shape[1] // 16

  def zrow(i, carry):
    for k in range(ncol):
      stage[i, pl.ds(k * 16, 16)] = zero16
    return carry

  lax.fori_loop(0, nrows, zrow, 0)


def _each_slice(s, ngroups, fn):
  base = ngroups // NTILE
  extra = ngroups - base * NTILE
  row0 = 8 * (s * base + jnp.minimum(s, extra))
  fn(row0, 8 * base)

  if extra:
    @pl.when(s < extra)
    def _():
      fn(row0 + 8 * base, 8)


def _zero_acc(acc, stage, s):
  def zero_slice(row0, nrows):
    nfull = nrows // CH
    rem = nrows - nfull * CH
    for j in range(nfull):
      pltpu.sync_copy(stage.at[pl.ds(0, CH)],
                      acc.at[pl.ds(row0 + j * CH, CH)])
    if rem:
      pltpu.sync_copy(stage.at[pl.ds(0, rem)],
                      acc.at[pl.ds(row0 + nfull * CH, rem)])

  _each_slice(s, acc.shape[0] // 8, zero_slice)


def _sc_agg_body(p_hbm, src_hbm, dst_hbm, out_hbm, *sc):
  sidx = sc[0:KBUF]
  didx = sc[KBUF:2 * KBUF]
  stage = sc[2 * KBUF:3 * KBUF]
  acc = sc[3 * KBUF]
  si = sc[3 * KBUF + 1:3 * KBUF + 1 + KBUF]
  sg = sc[3 * KBUF + 1 + KBUF:3 * KBUF + 1 + 2 * KBUF]
  ss = sc[3 * KBUF + 1 + 2 * KBUF:3 * KBUF + 1 + 3 * KBUF]
  c = lax.axis_index("c")
  s = lax.axis_index("s")

  _zero_stage(stage[0], CH)
  _zero_acc(acc, stage[0], s)
  plsc.subcore_barrier()

  base_chunk = (c * NTILE + s) * CPT
  dbase = c * N

  def group(g, carry):
    for b in range(KBUF):
      @pl.when(g > 0)
      def _():
        pltpu.make_async_copy(stage[b], acc.at[didx[b]], ss[b]).wait()

      eb = (base_chunk + g * KBUF + b) * CH
      pltpu.async_copy(src_hbm.at[pl.ds(eb, CH)], sidx[b], si[b])
      pltpu.async_copy(dst_hbm.at[pl.ds(eb, CH)], didx[b], si[b])
    for b in range(KBUF):
      eb = (base_chunk + g * KBUF + b) * CH
      pltpu.make_async_copy(src_hbm.at[pl.ds(eb, CH)], sidx[b], si[b]).wait()
      pltpu.make_async_copy(dst_hbm.at[pl.ds(eb, CH)], didx[b], si[b]).wait()
      for k in range(CH // 16):
        didx[b][pl.ds(k * 16, 16)] = didx[b][pl.ds(k * 16, 16)] - dbase
      pltpu.async_copy(p_hbm.at[sidx[b]], stage[b], sg[b])
    for b in range(KBUF):
      pltpu.make_async_copy(p_hbm.at[sidx[b]], stage[b], sg[b]).wait()
      pltpu.async_copy(stage[b], acc.at[didx[b]], ss[b], add=True)
    return carry

  lax.fori_loop(0, NGROUP, group, 0)
  for b in range(KBUF):
    pltpu.make_async_copy(stage[b], acc.at[didx[b]], ss[b]).wait()
  plsc.subcore_barrier()

  def writeout(row0, nrows):
    pltpu.sync_copy(acc.at[pl.ds(row0, nrows)],
                    out_hbm.at[pl.ds(dbase + row0, nrows)])

  _each_slice(s, N // 8, writeout)


@functools.lru_cache(maxsize=None)
def _sc_agg_kernel():
  return pl.kernel(
      _sc_agg_body,
      out_type=jax.ShapeDtypeStruct((NT, F), jnp.float32),
      mesh=_mesh(),
      scratch_types=(
          [pltpu.VMEM((CH,), jnp.int32)] * (2 * KBUF)
          + [pltpu.VMEM((CH, F), jnp.float32)] * KBUF
          + [pltpu.VMEM_SHARED((NACC, F), jnp.float32)]
          + [pltpu.SemaphoreType.DMA] * (3 * KBUF)
      ),
  )


def _sc_agg(p, src, dst):
  return _sc_agg_kernel()(p, src, dst)


def _sc_degree_body(dst_hbm, out_hbm, *sc):
  didx = sc[0:KBUF]
  stage = sc[KBUF]
  acc = sc[KBUF + 1]
  si = sc[KBUF + 2:2 * KBUF + 2]
  ss = sc[2 * KBUF + 2:3 * KBUF + 2]
  c = lax.axis_index("c")
  s = lax.axis_index("s")

  _zero_stage(stage, CH)
  _zero_acc(acc, stage, s)

  one0 = jnp.where(lax.iota(jnp.int32, 16) == 0, 1.0, 0.0).astype(jnp.float32)

  def orow(i, carry):
    stage[i, pl.ds(0, 16)] = one0
    return carry

  lax.fori_loop(0, CH, orow, 0)
  plsc.subcore_barrier()

  base_chunk = (c * NTILE + s) * CPT
  dbase = c * N

  def group(g, carry):
    for b in range(KBUF):
      @pl.when(g > 0)
      def _():
        pltpu.make_async_copy(stage, acc.at[didx[b]], ss[b]).wait()

      eb = (base_chunk + g * KBUF + b) * CH
      pltpu.async_copy(dst_hbm.at[pl.ds(eb, CH)], didx[b], si[b])
    for b in range(KBUF):
      eb = (base_chunk + g * KBUF + b) * CH
      pltpu.make_async_copy(dst_hbm.at[pl.ds(eb, CH)], didx[b], si[b]).wait()
      for k in range(CH // 16):
        didx[b][pl.ds(k * 16, 16)] = didx[b][pl.ds(k * 16, 16)] - dbase
      pltpu.async_copy(stage, acc.at[didx[b]], ss[b], add=True)
    return carry

  lax.fori_loop(0, NGROUP, group, 0)
  for b in range(KBUF):
    pltpu.make_async_copy(stage, acc.at[didx[b]], ss[b]).wait()
  plsc.subcore_barrier()

  def writeout(row0, nrows):
    pltpu.sync_copy(acc.at[pl.ds(row0, nrows)],
                    out_hbm.at[pl.ds(dbase + row0, nrows)])

  _each_slice(s, N // 8, writeout)


@functools.lru_cache(maxsize=None)
def _sc_degree_kernel():
  return pl.kernel(
      _sc_degree_body,
      out_type=jax.ShapeDtypeStruct((NT, 16), jnp.float32),
      mesh=_mesh(),
      scratch_types=(
          [pltpu.VMEM((CH,), jnp.int32)] * KBUF
          + [pltpu.VMEM((CH, 16), jnp.float32)]
          + [pltpu.VMEM_SHARED((NACC, 16), jnp.float32)]
          + [pltpu.SemaphoreType.DMA] * (2 * KBUF)
      ),
  )


def _sc_degree(dst):
  return _sc_degree_kernel()(dst)


def _prep_body(deg_ref, x_ref, dinv_ref, px_ref):
  d = deg_ref[:, 0:1] + 1.0
  dinvb = jnp.broadcast_to(lax.rsqrt(d), (RB, F))
  dinv_ref[...] = dinvb
  px_ref[...] = x_ref[...] * dinvb[None]


def _prep(degs, x_r):
  return pl.pallas_call(
      _prep_body,
      grid=(NT // RB,),
      in_specs=[
          pl.BlockSpec((RB, 16), lambda i: (i, 0)),
          pl.BlockSpec((T, RB, F), lambda i: (0, i, 0)),
      ],
      out_specs=[
          pl.BlockSpec((RB, F), lambda i: (i, 0)),
          pl.BlockSpec((T, RB, F), lambda i: (0, i, 0)),
      ],
      out_shape=[
          jax.ShapeDtypeStruct((NT, F), jnp.float32),
          jax.ShapeDtypeStruct((T, NT, F), jnp.float32),
      ],
  )(degs, x_r)


def _gates_body(sx_ref, sh_ref, px_ref, ph_ref, h_ref, dinv_ref, wz_ref,
                bz_ref, wr_ref, br_ref, z_ref, qrh_ref, aggx_ref):
  dinv = dinv_ref[...]
  aggx = dinv * (sx_ref[...] + px_ref[...])
  aggh = dinv * (sh_ref[...] + ph_ref[...])
  wz = wz_ref[...]
  wr = wr_ref[...]
  za = (jnp.dot(aggx, wz[:F], preferred_element_type=jnp.float32)
        + jnp.dot(aggh, wz[F:], preferred_element_type=jnp.float32)
        + bz_ref[...])
  ra = (jnp.dot(aggx, wr[:F], preferred_element_type=jnp.float32)
        + jnp.dot(aggh, wr[F:], preferred_element_type=jnp.float32)
        + br_ref[...])
  z = jax.nn.sigmoid(za)
  r = jax.nn.sigmoid(ra)
  z_ref[...] = z
  qrh_ref[...] = dinv * (r * h_ref[...])
  aggx_ref[...] = aggx


def _gates(sx, sh, px, ph, h, dinv_b, wz, bz, wr, br):
  row = pl.BlockSpec((RB, F), lambda i: (i, 0))
  wspec = pl.BlockSpec((2 * F, F), lambda i: (0, 0))
  bspec = pl.BlockSpec((1, F), lambda i: (0, 0))
  return pl.pallas_call(
      _gates_body,
      grid=(NT // RB,),
      in_specs=[row, row, row, row, row, row, wspec, bspec, wspec, bspec],
      out_specs=[row, row, row],
      out_shape=[jax.ShapeDtypeStruct((NT, F), jnp.float32)] * 3,
  )(sx, sh, px, ph, h, dinv_b, wz, bz, wr, br)


def _update_body(srh_ref, qrh_ref, aggx_ref, z_ref, h_ref, dinv_ref, wc_ref,
                 bc_ref, gamma_ref, beta_ref, h_out, ph_out):
  dinv = dinv_ref[...]
  aggrh = dinv * (srh_ref[...] + qrh_ref[...])
  wc = wc_ref[...]
  ca = (jnp.dot(aggx_ref[...], wc[:F], preferred_element_type=jnp.float32)
        + jnp.dot(aggrh, wc[F:], preferred_element_type=jnp.float32)
        + bc_ref[...])
  cand = jnp.tanh(ca)
  z = z_ref[...]
  hn = (1.0 - z) * h_ref[...] + z * cand
  mu = jnp.mean(hn, axis=1, keepdims=True)
  var = jnp.mean((hn - mu) ** 2, axis=1, keepdims=True)
  hln = (hn - mu) * lax.rsqrt(var + 1e-5) * gamma_ref[...] + beta_ref[...]
  h_out[...] = hln
  ph_out[...] = dinv * hln


def _update(srh, qrh, aggx, z, h, dinv_b, wc, bc, gamma, beta):
  row = pl.BlockSpec((RB, F), lambda i: (i, 0))
  wspec = pl.BlockSpec((2 * F, F), lambda i: (0, 0))
  bspec = pl.BlockSpec((1, F), lambda i: (0, 0))
  return pl.pallas_call(
      _update_body,
      grid=(NT // RB,),
      in_specs=[row, row, row, row, row, row, wspec, bspec, bspec, bspec],
      out_specs=[row, row],
      out_shape=[jax.ShapeDtypeStruct((NT, F), jnp.float32)] * 2,
  )(srh, qrh, aggx, z, h, dinv_b, wc, bc, gamma, beta)


def _pool_body(h_ref, out_ref):
  i = pl.program_id(0)

  @pl.when(i == 0)
  def _():
    out_ref[...] = jnp.zeros_like(out_ref)

  ssum = jnp.sum(h_ref[...], axis=0, keepdims=True) * (1.0 / N)
  blocks_per_sample = N // RB
  out_ref[0:1] += jnp.where(i < blocks_per_sample, ssum, 0.0)
  out_ref[1:2] += jnp.where(i >= blocks_per_sample, ssum, 0.0)


def _pool(h):
  return pl.pallas_call(
      _pool_body,
      grid=(NT // RB,),
      in_specs=[pl.BlockSpec((RB, F), lambda i: (i, 0))],
      out_specs=pl.BlockSpec((8, F), lambda i: (0, 0)),
      out_shape=jax.ShapeDtypeStruct((8, F), jnp.float32),
  )(h)


def kernel(x_sequence, edge_index, batch, Wz, bz, Wr, br, Wc, bc, gamma,
           beta):
  del batch
  ei = edge_index.astype(jnp.int32)
  pidx = jnp.arange(PAD, dtype=jnp.int32)
  pad_src = pidx % NT
  src = jnp.concatenate([ei[0], pad_src, ei[0] + N, pad_src])
  dst = jnp.concatenate(
      [ei[1], N + (pidx % 8), ei[1] + N, 2 * N + (pidx % 8)])
  x_r = jnp.transpose(x_sequence, (1, 0, 2, 3)).reshape(T, NT, F)

  degs = _sc_degree(dst)
  dinv_b, px = _prep(degs, x_r)

  bz2 = bz.reshape(1, F)
  br2 = br.reshape(1, F)
  bc2 = bc.reshape(1, F)
  gamma2 = gamma.reshape(1, F)
  beta2 = beta.reshape(1, F)

  h = jnp.zeros((NT, F), jnp.float32)
  ph = jnp.zeros((NT, F), jnp.float32)
  sh = jnp.zeros((NT, F), jnp.float32)
  for t in range(T):
    pxt = px[t]
    sxt = _sc_agg(pxt, src, dst)
    z, qrh, aggx = _gates(sxt, sh, pxt, ph, h, dinv_b, Wz, bz2, Wr, br2)
    srh = _sc_agg(qrh, src, dst)
    h, ph = _update(srh, qrh, aggx, z, h, dinv_b, Wc, bc2, gamma2, beta2)
    if t < T - 1:
      sh = _sc_agg(ph, src, dst)
  return _pool(h)[:NB]
